# baseline (speedup 1.0000x reference)
import jax
import jax.numpy as jnp
from jax.experimental import pallas as pl
from jax.experimental.pallas import tpu as pltpu

_NCHUNK = 8


def _dma_copy(in_ref, out_ref, sems):
    rows = in_ref.shape[0]
    chunk = rows // _NCHUNK
    for i in range(_NCHUNK):
        pltpu.make_async_copy(
            in_ref.at[pl.ds(i * chunk, chunk)],
            out_ref.at[pl.ds(i * chunk, chunk)],
            sems.at[i],
        ).start()
    for i in range(_NCHUNK):
        pltpu.make_async_copy(
            in_ref.at[pl.ds(i * chunk, chunk)],
            out_ref.at[pl.ds(i * chunk, chunk)],
            sems.at[i],
        ).wait()


def kernel(table):
    n, d = table.shape
    wide = table.reshape(n // 4, d * 4)
    out = pl.pallas_call(
        _dma_copy,
        in_specs=[pl.BlockSpec(memory_space=pl.ANY)],
        out_specs=pl.BlockSpec(memory_space=pl.ANY),
        out_shape=jax.ShapeDtypeStruct(wide.shape, wide.dtype),
        scratch_shapes=[pltpu.SemaphoreType.DMA((_NCHUNK,))],
    )(wide)
    return out.reshape(n, d)


# speedup vs baseline: 4.5004x; 4.5004x over previous
import jax
import jax.numpy as jnp
from jax.experimental import pallas as pl
from jax.experimental.pallas import tpu as pltpu


def _copy_block(in_ref, out_ref):
    out_ref[...] = in_ref[...]


def kernel(table):
    n, d = table.shape
    wide = table.reshape(n // 4, d * 4)
    rows = wide.shape[0]
    block = 10000
    out = pl.pallas_call(
        _copy_block,
        grid=(rows // block,),
        in_specs=[pl.BlockSpec((block, d * 4), lambda i: (i, 0))],
        out_specs=pl.BlockSpec((block, d * 4), lambda i: (i, 0)),
        out_shape=jax.ShapeDtypeStruct(wide.shape, wide.dtype),
    )(wide)
    return out.reshape(n, d)
